# Initial kernel scaffold; baseline (speedup 1.0000x reference)
import jax
import jax.numpy as jnp
from jax.experimental import pallas as pl


def kernel(edge_index, edge_attr, params):
    raise NotImplementedError("write your pallas kernel here")


# speedup vs baseline: 3.9782x; 3.9782x over previous
import dataclasses
import functools

import jax
import jax.numpy as jnp
from jax import lax
from jax.experimental import pallas as pl
from jax.experimental.pallas import tpu as pltpu
from jax.experimental.pallas import tpu_sc as plsc

N = 10000
E = 320000
H = 64
NC = 2
NS = 16
NW = NC * NS
NPAD = 10240
EPT = E // NW
CH = 2000
UNR = 5
GCHUNK = 80
TW = 128
EBLK = 3200
NEB = E // EBLK
NBLK = 2048
NNB = NPAD // NBLK

_F32 = jnp.float32
_HI = lax.Precision.HIGHEST


def _mm(a, b):
    return lax.dot_general(a, b, (((a.ndim - 1,), (0,)), ((), ())),
                           preferred_element_type=_F32, precision=_HI)


def _sc_mesh():
    return plsc.VectorSubcoreMesh(core_axis_name="c", subcore_axis_name="s")


def _sc_params():
    cp = pltpu.CompilerParams()
    if "needs_layout_passes" in pltpu.CompilerParams.__dataclass_fields__:
        cp = dataclasses.replace(cp, needs_layout_passes=False)
    return cp


def _sc_count(dst):
    def body(dst_hbm, out_hbm, dbuf, acc):
        c = lax.axis_index("c")
        s = lax.axis_index("s")
        w = c * NS + s

        @pl.loop(0, NPAD // 16)
        def _z(i):
            acc[pl.ds(i * 16, 16)] = jnp.zeros((16,), _F32)

        ones = jnp.ones((16,), _F32)
        base = w * EPT

        @pl.loop(0, EPT // CH)
        def _chunk(i):
            pltpu.sync_copy(dst_hbm.at[pl.ds(base + i * CH, CH)], dbuf)

            @pl.loop(0, CH // 16 // UNR)
            def _g(g):
                for u in range(UNR):
                    iv = dbuf[pl.ds((g * UNR + u) * 16, 16)]
                    plsc.addupdate_scatter(acc, [iv], ones)

        pltpu.sync_copy(acc, out_hbm.at[w])

    run = functools.partial(
        pl.kernel, body,
        out_type=jax.ShapeDtypeStruct((NW, NPAD), _F32),
        mesh=_sc_mesh(),
        compiler_params=_sc_params(),
        scratch_types=[
            pltpu.VMEM((CH,), jnp.int32),
            pltpu.VMEM((NPAD,), _F32),
        ])()
    return run(dst)


def _sc_colscat(yt, src, dst):
    def body(yt_hbm, src_hbm, dst_hbm, out_hbm, sbuf, dbuf, y0c, y1c, a0, a1):
        c = lax.axis_index("c")
        s = lax.axis_index("s")
        f0 = 2 * (c * NS + s)

        @pl.loop(0, NPAD // 16)
        def _z(i):
            sl = pl.ds(i * 16, 16)
            a0[sl] = jnp.zeros((16,), _F32)
            a1[sl] = jnp.zeros((16,), _F32)

        pltpu.sync_copy(yt_hbm.at[f0], y0c)
        pltpu.sync_copy(yt_hbm.at[f0 + 1], y1c)

        @pl.loop(0, E // CH)
        def _chunk(i):
            pltpu.sync_copy(src_hbm.at[pl.ds(i * CH, CH)], sbuf)
            pltpu.sync_copy(dst_hbm.at[pl.ds(i * CH, CH)], dbuf)

            @pl.loop(0, CH // 16 // UNR)
            def _g(g):
                for u in range(UNR):
                    sl = pl.ds((g * UNR + u) * 16, 16)
                    siv = sbuf[sl]
                    div = dbuf[sl]
                    plsc.addupdate_scatter(a0, [div], plsc.load_gather(y0c, [siv]))
                    plsc.addupdate_scatter(a1, [div], plsc.load_gather(y1c, [siv]))

        pltpu.sync_copy(a0, out_hbm.at[f0])
        pltpu.sync_copy(a1, out_hbm.at[f0 + 1])

    run = functools.partial(
        pl.kernel, body,
        out_type=jax.ShapeDtypeStruct((H, NPAD), _F32),
        mesh=_sc_mesh(),
        compiler_params=_sc_params(),
        scratch_types=[
            pltpu.VMEM((CH,), jnp.int32),
            pltpu.VMEM((CH,), jnp.int32),
            pltpu.VMEM((NPAD,), _F32),
            pltpu.VMEM((NPAD,), _F32),
            pltpu.VMEM((NPAD,), _F32),
            pltpu.VMEM((NPAD,), _F32),
        ])()
    return run(yt, src, dst)


def _sc_gather_pairs(a128, b128, src, dst):
    def body(a_hbm, b_hbm, src_hbm, dst_hbm, out_hbm, sidx, didx, ra, rb, r64, sem, sem2):
        c = lax.axis_index("c")
        s = lax.axis_index("s")
        base = (c * NS + s) * EPT

        @pl.loop(0, EPT // GCHUNK)
        def _chunk(i):
            off = base + i * GCHUNK
            pltpu.sync_copy(src_hbm.at[pl.ds(off, GCHUNK)], sidx)
            pltpu.sync_copy(dst_hbm.at[pl.ds(off, GCHUNK)], didx)
            h1 = pltpu.async_copy(a_hbm.at[sidx], ra, sem)
            h2 = pltpu.async_copy(b_hbm.at[didx], rb, sem2)
            h1.wait()
            h2.wait()

            @pl.loop(0, GCHUNK)
            def _row(r):
                for j in range(H // 16):
                    sl = pl.ds(j * 16, 16)
                    r64[r, sl] = ra[r, sl] + rb[r, sl]

            pltpu.sync_copy(r64, out_hbm.at[pl.ds(off, GCHUNK)])

    run = functools.partial(
        pl.kernel, body,
        out_type=jax.ShapeDtypeStruct((E, H), _F32),
        mesh=_sc_mesh(),
        scratch_types=[
            pltpu.VMEM((GCHUNK,), jnp.int32),
            pltpu.VMEM((GCHUNK,), jnp.int32),
            pltpu.VMEM((GCHUNK, TW), _F32),
            pltpu.VMEM((GCHUNK, TW), _F32),
            pltpu.VMEM((GCHUNK, H), _F32),
            pltpu.SemaphoreType.DMA,
            pltpu.SemaphoreType.DMA,
        ])()
    return run(a128, b128, src, dst)


def _tc_prep(cnt, x0row, gcn_w0):
    def body(cnt_r, x0, w, di_o, y0t_o):
        deg = jnp.sum(cnt_r[...], axis=0, keepdims=True) + 1.0
        di = lax.rsqrt(jnp.maximum(deg, 1.0))
        w0 = _mm(x0[...], w[...])
        di_o[...] = di
        y0t_o[...] = jnp.transpose(w0) * di

    return pl.pallas_call(
        body,
        out_shape=[jax.ShapeDtypeStruct((1, NPAD), _F32),
                   jax.ShapeDtypeStruct((H, NPAD), _F32)],
    )(cnt, x0row, gcn_w0)


def _lanespec():
    return pl.BlockSpec((H, NBLK), lambda i: (0, i))


def _dispec():
    return pl.BlockSpec((1, NBLK), lambda i: (0, i))


def _rowspec():
    return pl.BlockSpec((NBLK, H), lambda i: (i, 0))


def _vspec():
    return pl.BlockSpec((1, H), lambda i: (0, 0))


def _wspec():
    return pl.BlockSpec((H, H), lambda i: (0, 0))


def _tc_node_a(accT, yT, diT, gcnb):
    def body(a_r, y_r, di, b_r, oo, so, qo):
        lane = (lax.broadcasted_iota(jnp.int32, (H, NBLK), 1)
                + pl.program_id(0) * NBLK)
        out = di[...] * (a_r[...] + y_r[...]) + jnp.transpose(b_r[...])
        out = jnp.where(lane < N, out, 0.0)
        oo[...] = out

        @pl.when(pl.program_id(0) == 0)
        def _init():
            so[...] = jnp.zeros_like(so)
            qo[...] = jnp.zeros_like(qo)

        so[...] += jnp.transpose(jnp.sum(out, axis=1, keepdims=True))
        qo[...] += jnp.transpose(jnp.sum(out * out, axis=1, keepdims=True))

    return pl.pallas_call(
        body,
        grid=(NNB,),
        in_specs=[_lanespec(), _lanespec(), _dispec(), _vspec()],
        out_specs=[_lanespec(), _vspec(), _vspec()],
        out_shape=[jax.ShapeDtypeStruct((H, NPAD), _F32),
                   jax.ShapeDtypeStruct((1, H), _F32),
                   jax.ShapeDtypeStruct((1, H), _F32)],
    )(accT, yT, diT, gcnb)


def _tc_node_b(outT, av, bv, na, nb, diT, wn=None):
    if wn is not None:
        def body(o_r, a_r, b_r, na_r, nb_r, di, wn_r, ao, bo, yo, gs):
            lane = (lax.broadcasted_iota(jnp.int32, (H, NBLK), 1)
                    + pl.program_id(0) * NBLK)
            x = jnp.maximum(o_r[...] * jnp.transpose(a_r[...])
                            + jnp.transpose(b_r[...]), 0.0)
            x = jnp.where(lane < N, x, 0.0)
            ao[...] = jnp.transpose(_mm(jnp.transpose(na_r[...]), x))
            bo[...] = jnp.transpose(_mm(jnp.transpose(nb_r[...]), x))
            yo[...] = _mm(jnp.transpose(wn_r[...]), x) * di[...]

            @pl.when(pl.program_id(0) == 0)
            def _init():
                gs[...] = jnp.zeros_like(gs)

            gs[...] += jnp.transpose(jnp.sum(x, axis=1, keepdims=True))

        return pl.pallas_call(
            body,
            grid=(NNB,),
            in_specs=[_lanespec(), _vspec(), _vspec(), _wspec(), _wspec(),
                      _dispec(), _wspec()],
            out_specs=[_rowspec(), _rowspec(), _lanespec(), _vspec()],
            out_shape=[jax.ShapeDtypeStruct((NPAD, H), _F32),
                       jax.ShapeDtypeStruct((NPAD, H), _F32),
                       jax.ShapeDtypeStruct((H, NPAD), _F32),
                       jax.ShapeDtypeStruct((1, H), _F32)],
        )(outT, av, bv, na, nb, diT, wn)

    def body2(o_r, a_r, b_r, na_r, nb_r, di, ao, bo, gs):
        lane = (lax.broadcasted_iota(jnp.int32, (H, NBLK), 1)
                + pl.program_id(0) * NBLK)
        x = jnp.maximum(o_r[...] * jnp.transpose(a_r[...])
                        + jnp.transpose(b_r[...]), 0.0)
        x = jnp.where(lane < N, x, 0.0)
        ao[...] = jnp.transpose(_mm(jnp.transpose(na_r[...]), x))
        bo[...] = jnp.transpose(_mm(jnp.transpose(nb_r[...]), x))

        @pl.when(pl.program_id(0) == 0)
        def _init():
            gs[...] = jnp.zeros_like(gs)

        gs[...] += jnp.transpose(jnp.sum(x, axis=1, keepdims=True))

    return pl.pallas_call(
        body2,
        grid=(NNB,),
        in_specs=[_lanespec(), _vspec(), _vspec(), _wspec(), _wspec(), _dispec()],
        out_specs=[_rowspec(), _rowspec(), _vspec()],
        out_shape=[jax.ShapeDtypeStruct((NPAD, H), _F32),
                   jax.ShapeDtypeStruct((NPAD, H), _F32),
                   jax.ShapeDtypeStruct((1, H), _F32)],
    )(outT, av, bv, na, nb, diT)


def _tc_vhead(gsum, w1, b1, w2row, b2):
    def body(gs, w1_r, b1_r, w2_r, b2_r, vo):
        gv = gs[...] * (1.0 / N)
        t1 = jnp.maximum(_mm(gv, w1_r[...]) + b1_r[...], 0.0)
        t2 = jnp.sum(t1 * w2_r[...], axis=1, keepdims=True) + b2_r[...]
        vo[...] = jnp.tanh(t2)

    return pl.pallas_call(
        body,
        out_shape=jax.ShapeDtypeStruct((1, 1), _F32),
    )(gsum, w1, b1, w2row, b2)


def _tc_edge(gath, esrc, k, cvec, ab=None):
    ecols = esrc.shape[1]

    def make_body(with_norm):
        def body(*refs):
            if with_norm:
                g_r, e_r, k_r, c_r, a_r, b_r, co, so, qo = refs
                e = jnp.maximum(e_r[...] * a_r[...] + b_r[...], 0.0)
            else:
                g_r, e_r, k_r, c_r, co, so, qo = refs
                e = e_r[...]
            comb = g_r[...] + _mm(e, k_r[...]) + c_r[...]
            co[...] = comb

            @pl.when(pl.program_id(0) == 0)
            def _init():
                so[...] = jnp.zeros_like(so)
                qo[...] = jnp.zeros_like(qo)

            so[...] += jnp.sum(comb, axis=0, keepdims=True)
            qo[...] += jnp.sum(comb * comb, axis=0, keepdims=True)
        return body

    in_specs = [
        pl.BlockSpec((EBLK, H), lambda i: (i, 0)),
        pl.BlockSpec((EBLK, ecols), lambda i: (i, 0)),
        pl.BlockSpec(k.shape, lambda i: (0, 0)),
        pl.BlockSpec((1, H), lambda i: (0, 0)),
    ]
    args = [gath, esrc, k, cvec]
    if ab is not None:
        in_specs += [pl.BlockSpec((1, H), lambda i: (0, 0))] * 2
        args += [ab[0], ab[1]]

    return pl.pallas_call(
        make_body(ab is not None),
        grid=(NEB,),
        in_specs=in_specs,
        out_specs=[pl.BlockSpec((EBLK, H), lambda i: (i, 0)),
                   pl.BlockSpec((1, H), lambda i: (0, 0)),
                   pl.BlockSpec((1, H), lambda i: (0, 0))],
        out_shape=[jax.ShapeDtypeStruct((E, H), _F32),
                   jax.ShapeDtypeStruct((1, H), _F32),
                   jax.ShapeDtypeStruct((1, H), _F32)],
    )(*args)


def _tc_head(comb, avec, bvec, w1, b1, w2row, b2):
    def body(c_r, a_r, b_r, w1_r, b1_r, w2_r, b2_r, so):
        e = jnp.maximum(c_r[...] * a_r[...] + b_r[...], 0.0)
        h = jnp.maximum(_mm(e, w1_r[...]) + b1_r[...], 0.0)
        so[...] = (jnp.sum(h * w2_r[...], axis=1) + b2_r[0, 0]).reshape(1, 1, EBLK)

    return pl.pallas_call(
        body,
        grid=(NEB,),
        in_specs=[
            pl.BlockSpec((EBLK, H), lambda i: (i, 0)),
            pl.BlockSpec((1, H), lambda i: (0, 0)),
            pl.BlockSpec((1, H), lambda i: (0, 0)),
            pl.BlockSpec((H, 4 * H), lambda i: (0, 0)),
            pl.BlockSpec((1, 4 * H), lambda i: (0, 0)),
            pl.BlockSpec((1, 4 * H), lambda i: (0, 0)),
            pl.BlockSpec((1, 1), lambda i: (0, 0)),
        ],
        out_specs=pl.BlockSpec((1, 1, EBLK), lambda i: (i, 0, 0)),
        out_shape=jax.ShapeDtypeStruct((NEB, 1, EBLK), _F32),
    )(comb, avec, bvec, w1, b1, w2row, b2)


def _tc_softmax(scores2d):
    def body(s_r, p_r):
        s = s_r[...]
        m = jnp.max(s)
        ex = jnp.exp(s - m)
        p_r[...] = ex / jnp.sum(ex)

    return pl.pallas_call(
        body,
        out_shape=jax.ShapeDtypeStruct(scores2d.shape, _F32),
    )(scores2d)


def _fold(a, b):
    return jnp.dot(a, b, precision=_HI)


def kernel(edge_index, edge_attr, params):
    src = edge_index[0].astype(jnp.int32)
    dst = edge_index[1].astype(jnp.int32)
    p = params
    l0, l1 = p["layer0"], p["layer1"]

    cbwa0, cbwb0 = l0["cb_W"][:H], l0["cb_W"][H:]
    cbwa1, cbwb1 = l1["cb_W"][:H], l1["cb_W"][H:]
    na0 = _fold(l0["np_W"][:H], cbwa0)
    nb0 = _fold(l0["np_W"][H:], cbwa0)
    na1 = _fold(l1["np_W"][:H], cbwa1)
    nb1 = _fold(l1["np_W"][H:], cbwa1)
    k0 = _fold(l0["ep_W"], cbwb0)
    ek0 = _fold(p["ee_W"], k0)
    k1 = _fold(l1["ep_W"], cbwb1)
    c0 = (_fold(l0["np_b"], cbwa0) + _fold(p["ee_b"], k0)
          + _fold(l0["ep_b"], cbwb0) + l0["cb_b"]).reshape(1, H)
    c1 = (_fold(l1["np_b"], cbwa1) + _fold(l1["ep_b"], cbwb1)
          + l1["cb_b"]).reshape(1, H)
    x0row = (p["ne_W"][0] + p["ne_b"]).reshape(1, H)

    def _pad(t):
        return jnp.pad(t, ((0, 0), (0, TW - H)))

    def _bn_fold(s, q, cnt_n, gamma, betav):
        m = s / cnt_n
        var = jnp.maximum(q / cnt_n - m * m, 0.0)
        istd = lax.rsqrt(var + 1e-5)
        av = istd * gamma.reshape(1, H)
        return av, betav.reshape(1, H) - m * av

    cnt = _sc_count(dst)
    diT, y0T = _tc_prep(cnt, x0row, l0["gcn_W"])

    accT0 = _sc_colscat(y0T, src, dst)
    outT0, ns0, nq0 = _tc_node_a(accT0, y0T, diT, l0["gcn_b"].reshape(1, H))
    nav0, nbv0 = _bn_fold(ns0, nq0, N, l0["gcn_g"], l0["gcn_beta"])
    a0t, b0t, y1T, _ = _tc_node_b(outT0, nav0, nbv0, na0, nb0, diT,
                                  wn=l1["gcn_W"])

    accT1 = _sc_colscat(y1T, src, dst)
    gath0 = _sc_gather_pairs(_pad(a0t), _pad(b0t), src, dst)
    comb0, s0, q0 = _tc_edge(gath0, edge_attr, ek0, c0)

    outT1, ns1, nq1 = _tc_node_a(accT1, y1T, diT, l1["gcn_b"].reshape(1, H))
    nav1, nbv1 = _bn_fold(ns1, nq1, N, l1["gcn_g"], l1["gcn_beta"])
    a1t, b1t, g1 = _tc_node_b(outT1, nav1, nbv1, na1, nb1, diT)
    v = _tc_vhead(g1, p["vh_W1"], p["vh_b1"].reshape(1, H // 2),
                  p["vh_W2"].reshape(1, H // 2), p["vh_b2"].reshape(1, 1))

    av0, bv0 = _bn_fold(s0, q0, E, l0["eb_g"], l0["eb_beta"])
    gath1 = _sc_gather_pairs(_pad(a1t), _pad(b1t), src, dst)
    comb1, s1, q1 = _tc_edge(gath1, comb0, k1, c1, ab=(av0, bv0))

    av1, bv1 = _bn_fold(s1, q1, E, l1["eb_g"], l1["eb_beta"])
    scores = _tc_head(comb1, av1, bv1, p["ph_W1"], p["ph_b1"].reshape(1, 4 * H),
                      p["ph_W2"].reshape(1, 4 * H), p["ph_b2"].reshape(1, 1))
    policy = _tc_softmax(scores.reshape(2500, 128)).reshape(E)
    return policy, v.reshape(1)


# speedup vs baseline: 6.8957x; 1.7334x over previous
import dataclasses
import functools

import jax
import jax.numpy as jnp
from jax import lax
from jax.experimental import pallas as pl
from jax.experimental.pallas import tpu as pltpu
from jax.experimental.pallas import tpu_sc as plsc

N = 10000
E = 320000
H = 64
NC = 2
NS = 16
NW = NC * NS
NPAD = 10240
EPT = E // NW
CH = 2000
UNR = 5
GCHUNK = 80
TW = 128
EBLK = 3200
NEB = E // EBLK
NBLK = 2048
NNB = NPAD // NBLK

_F32 = jnp.float32
_HI = lax.Precision.HIGHEST


def _mm(a, b):
    return lax.dot_general(a, b, (((a.ndim - 1,), (0,)), ((), ())),
                           preferred_element_type=_F32, precision=_HI)


def _mmd(a, b):
    return lax.dot_general(a, b, (((a.ndim - 1,), (0,)), ((), ())),
                           preferred_element_type=_F32)


def _sc_mesh():
    return plsc.VectorSubcoreMesh(core_axis_name="c", subcore_axis_name="s")


def _sc_params():
    cp = pltpu.CompilerParams()
    if "needs_layout_passes" in pltpu.CompilerParams.__dataclass_fields__:
        cp = dataclasses.replace(cp, needs_layout_passes=False)
    return cp


def _sc_count(dst):
    def body(dst_hbm, out_hbm, dbuf, acc):
        c = lax.axis_index("c")
        s = lax.axis_index("s")
        w = c * NS + s

        @pl.loop(0, NPAD // 16)
        def _z(i):
            acc[pl.ds(i * 16, 16)] = jnp.zeros((16,), _F32)

        ones = jnp.ones((16,), _F32)
        base = w * EPT

        @pl.loop(0, EPT // CH)
        def _chunk(i):
            pltpu.sync_copy(dst_hbm.at[pl.ds(base + i * CH, CH)], dbuf)

            @pl.loop(0, CH // 16 // UNR)
            def _g(g):
                for u in range(UNR):
                    iv = dbuf[pl.ds((g * UNR + u) * 16, 16)]
                    plsc.addupdate_scatter(acc, [iv], ones)

        pltpu.sync_copy(acc, out_hbm.at[w])

    run = functools.partial(
        pl.kernel, body,
        out_type=jax.ShapeDtypeStruct((NW, NPAD), _F32),
        mesh=_sc_mesh(),
        compiler_params=_sc_params(),
        scratch_types=[
            pltpu.VMEM((CH,), jnp.int32),
            pltpu.VMEM((NPAD,), _F32),
        ])()
    return run(dst)


def _sc_colscat(yt, src, dst):
    NCH = E // CH

    def body(yt_hbm, src_hbm, dst_hbm, out_hbm,
             sb0, db0, sb1, db1, y0c, y1c, a0, a1, ss0, sd0, ss1, sd1):
        c = lax.axis_index("c")
        s = lax.axis_index("s")
        f0 = 2 * (c * NS + s)

        @pl.loop(0, NPAD // 16)
        def _z(i):
            sl = pl.ds(i * 16, 16)
            a0[sl] = jnp.zeros((16,), _F32)
            a1[sl] = jnp.zeros((16,), _F32)

        pltpu.sync_copy(yt_hbm.at[f0], y0c)
        pltpu.sync_copy(yt_hbm.at[f0 + 1], y1c)
        bufs = ((sb0, db0, ss0, sd0), (sb1, db1, ss1, sd1))

        def launch(i, b):
            sb, db, ss, sd = bufs[b]
            pltpu.async_copy(src_hbm.at[pl.ds(i * CH, CH)], sb, ss)
            pltpu.async_copy(dst_hbm.at[pl.ds(i * CH, CH)], db, sd)

        def process(b):
            sb, db, ss, sd = bufs[b]
            pltpu.make_async_copy(src_hbm.at[pl.ds(0, CH)], sb, ss).wait()
            pltpu.make_async_copy(dst_hbm.at[pl.ds(0, CH)], db, sd).wait()

            @pl.loop(0, CH // 16 // UNR)
            def _g(g):
                for u in range(UNR):
                    sl = pl.ds((g * UNR + u) * 16, 16)
                    siv = sb[sl]
                    div = db[sl]
                    plsc.addupdate_scatter(a0, [div], plsc.load_gather(y0c, [siv]))
                    plsc.addupdate_scatter(a1, [div], plsc.load_gather(y1c, [siv]))

        launch(0, 0)
        launch(1, 1)

        @pl.loop(0, (NCH - 2) // 2)
        def _pair(j):
            i = 2 * j
            process(0)
            launch(i + 2, 0)
            process(1)
            launch(i + 3, 1)

        process(0)
        process(1)
        pltpu.sync_copy(a0, out_hbm.at[f0])
        pltpu.sync_copy(a1, out_hbm.at[f0 + 1])

    run = functools.partial(
        pl.kernel, body,
        out_type=jax.ShapeDtypeStruct((H, NPAD), _F32),
        mesh=_sc_mesh(),
        compiler_params=_sc_params(),
        scratch_types=[
            pltpu.VMEM((CH,), jnp.int32),
            pltpu.VMEM((CH,), jnp.int32),
            pltpu.VMEM((CH,), jnp.int32),
            pltpu.VMEM((CH,), jnp.int32),
            pltpu.VMEM((NPAD,), _F32),
            pltpu.VMEM((NPAD,), _F32),
            pltpu.VMEM((NPAD,), _F32),
            pltpu.VMEM((NPAD,), _F32),
            pltpu.SemaphoreType.DMA,
            pltpu.SemaphoreType.DMA,
            pltpu.SemaphoreType.DMA,
            pltpu.SemaphoreType.DMA,
        ])()
    return run(yt, src, dst)


def _sc_gather_pairs(a128, b128, src, dst):
    NCH = EPT // GCHUNK

    def body(a_hbm, b_hbm, src_hbm, dst_hbm, out_hbm, sidx, didx,
             ra0, rb0, r0, ra1, rb1, r1, sa0, sb0, sw0, sa1, sb1, sw1):
        c = lax.axis_index("c")
        s = lax.axis_index("s")
        base = (c * NS + s) * EPT
        pltpu.sync_copy(src_hbm.at[pl.ds(base, EPT)], sidx)
        pltpu.sync_copy(dst_hbm.at[pl.ds(base, EPT)], didx)
        bufs = ((ra0, rb0, r0, sa0, sb0, sw0), (ra1, rb1, r1, sa1, sb1, sw1))

        def launch(i, b):
            ra, rb, _, sa, sb, _ = bufs[b]
            pltpu.async_copy(a_hbm.at[sidx.at[pl.ds(i * GCHUNK, GCHUNK)]], ra, sa)
            pltpu.async_copy(b_hbm.at[didx.at[pl.ds(i * GCHUNK, GCHUNK)]], rb, sb)

        def process(i, b, first):
            ra, rb, r64, sa, sb, sw = bufs[b]
            pltpu.make_async_copy(a_hbm.at[sidx.at[pl.ds(0, GCHUNK)]], ra, sa).wait()
            pltpu.make_async_copy(b_hbm.at[didx.at[pl.ds(0, GCHUNK)]], rb, sb).wait()
            if not first:
                pltpu.make_async_copy(r64, out_hbm.at[pl.ds(0, GCHUNK)], sw).wait()

            @pl.loop(0, GCHUNK // 4)
            def _row(q):
                for u in range(4):
                    r = q * 4 + u
                    for j in range(H // 16):
                        sl = pl.ds(j * 16, 16)
                        r64[r, sl] = ra[r, sl] + rb[r, sl]

            pltpu.async_copy(r64, out_hbm.at[pl.ds(base + i * GCHUNK, GCHUNK)], sw)

        launch(0, 0)
        launch(1, 1)
        process(0, 0, True)
        launch(2, 0)
        process(1, 1, True)
        launch(3, 1)

        @pl.loop(0, (NCH - 4) // 2)
        def _pair(j):
            i = 2 * j + 2
            process(i, 0, False)
            launch(i + 2, 0)
            process(i + 1, 1, False)
            launch(i + 3, 1)

        process(NCH - 3, 0, False)
        launch(NCH - 1, 0)
        process(NCH - 2, 1, False)
        process(NCH - 1, 0, False)
        pltpu.make_async_copy(r0, out_hbm.at[pl.ds(0, GCHUNK)], sw0).wait()
        pltpu.make_async_copy(r1, out_hbm.at[pl.ds(0, GCHUNK)], sw1).wait()

    run = functools.partial(
        pl.kernel, body,
        out_type=jax.ShapeDtypeStruct((E, H), _F32),
        mesh=_sc_mesh(),
        scratch_types=[
            pltpu.VMEM((EPT,), jnp.int32),
            pltpu.VMEM((EPT,), jnp.int32),
            pltpu.VMEM((GCHUNK, TW), _F32),
            pltpu.VMEM((GCHUNK, TW), _F32),
            pltpu.VMEM((GCHUNK, H), _F32),
            pltpu.VMEM((GCHUNK, TW), _F32),
            pltpu.VMEM((GCHUNK, TW), _F32),
            pltpu.VMEM((GCHUNK, H), _F32),
            pltpu.SemaphoreType.DMA,
            pltpu.SemaphoreType.DMA,
            pltpu.SemaphoreType.DMA,
            pltpu.SemaphoreType.DMA,
            pltpu.SemaphoreType.DMA,
            pltpu.SemaphoreType.DMA,
        ])()
    return run(a128, b128, src, dst)


def _tc_prep(cnt, x0row, gcn_w0):
    def body(cnt_r, x0, w, di_o, y0t_o):
        deg = jnp.sum(cnt_r[...], axis=0, keepdims=True) + 1.0
        di = lax.rsqrt(jnp.maximum(deg, 1.0))
        w0 = _mm(x0[...], w[...])
        di_o[...] = di
        y0t_o[...] = jnp.transpose(w0) * di

    return pl.pallas_call(
        body,
        out_shape=[jax.ShapeDtypeStruct((1, NPAD), _F32),
                   jax.ShapeDtypeStruct((H, NPAD), _F32)],
    )(cnt, x0row, gcn_w0)


def _lanespec():
    return pl.BlockSpec((H, NBLK), lambda i: (0, i))


def _dispec():
    return pl.BlockSpec((1, NBLK), lambda i: (0, i))


def _rowspec():
    return pl.BlockSpec((NBLK, H), lambda i: (i, 0))


def _vspec():
    return pl.BlockSpec((1, H), lambda i: (0, 0))


def _wspec():
    return pl.BlockSpec((H, H), lambda i: (0, 0))


def _tc_node_a(accT, yT, diT, gcnb):
    def body(a_r, y_r, di, b_r, oo, so, qo):
        lane = (lax.broadcasted_iota(jnp.int32, (H, NBLK), 1)
                + pl.program_id(0) * NBLK)
        out = di[...] * (a_r[...] + y_r[...]) + jnp.transpose(b_r[...])
        out = jnp.where(lane < N, out, 0.0)
        oo[...] = out

        @pl.when(pl.program_id(0) == 0)
        def _init():
            so[...] = jnp.zeros_like(so)
            qo[...] = jnp.zeros_like(qo)

        so[...] += jnp.transpose(jnp.sum(out, axis=1, keepdims=True))
        qo[...] += jnp.transpose(jnp.sum(out * out, axis=1, keepdims=True))

    return pl.pallas_call(
        body,
        grid=(NNB,),
        in_specs=[_lanespec(), _lanespec(), _dispec(), _vspec()],
        out_specs=[_lanespec(), _vspec(), _vspec()],
        out_shape=[jax.ShapeDtypeStruct((H, NPAD), _F32),
                   jax.ShapeDtypeStruct((1, H), _F32),
                   jax.ShapeDtypeStruct((1, H), _F32)],
    )(accT, yT, diT, gcnb)


def _tc_node_b(outT, av, bv, na, nb, diT, wn=None):
    if wn is not None:
        def body(o_r, a_r, b_r, na_r, nb_r, di, wn_r, ao, bo, yo, gs):
            lane = (lax.broadcasted_iota(jnp.int32, (H, NBLK), 1)
                    + pl.program_id(0) * NBLK)
            x = jnp.maximum(o_r[...] * jnp.transpose(a_r[...])
                            + jnp.transpose(b_r[...]), 0.0)
            x = jnp.where(lane < N, x, 0.0)
            ao[...] = jnp.transpose(_mm(jnp.transpose(na_r[...]), x))
            bo[...] = jnp.transpose(_mm(jnp.transpose(nb_r[...]), x))
            yo[...] = _mm(jnp.transpose(wn_r[...]), x) * di[...]

            @pl.when(pl.program_id(0) == 0)
            def _init():
                gs[...] = jnp.zeros_like(gs)

            gs[...] += jnp.transpose(jnp.sum(x, axis=1, keepdims=True))

        return pl.pallas_call(
            body,
            grid=(NNB,),
            in_specs=[_lanespec(), _vspec(), _vspec(), _wspec(), _wspec(),
                      _dispec(), _wspec()],
            out_specs=[_rowspec(), _rowspec(), _lanespec(), _vspec()],
            out_shape=[jax.ShapeDtypeStruct((NPAD, H), _F32),
                       jax.ShapeDtypeStruct((NPAD, H), _F32),
                       jax.ShapeDtypeStruct((H, NPAD), _F32),
                       jax.ShapeDtypeStruct((1, H), _F32)],
        )(outT, av, bv, na, nb, diT, wn)

    def body2(o_r, a_r, b_r, na_r, nb_r, di, ao, bo, gs):
        lane = (lax.broadcasted_iota(jnp.int32, (H, NBLK), 1)
                + pl.program_id(0) * NBLK)
        x = jnp.maximum(o_r[...] * jnp.transpose(a_r[...])
                        + jnp.transpose(b_r[...]), 0.0)
        x = jnp.where(lane < N, x, 0.0)
        ao[...] = jnp.transpose(_mm(jnp.transpose(na_r[...]), x))
        bo[...] = jnp.transpose(_mm(jnp.transpose(nb_r[...]), x))

        @pl.when(pl.program_id(0) == 0)
        def _init():
            gs[...] = jnp.zeros_like(gs)

        gs[...] += jnp.transpose(jnp.sum(x, axis=1, keepdims=True))

    return pl.pallas_call(
        body2,
        grid=(NNB,),
        in_specs=[_lanespec(), _vspec(), _vspec(), _wspec(), _wspec(), _dispec()],
        out_specs=[_rowspec(), _rowspec(), _vspec()],
        out_shape=[jax.ShapeDtypeStruct((NPAD, H), _F32),
                   jax.ShapeDtypeStruct((NPAD, H), _F32),
                   jax.ShapeDtypeStruct((1, H), _F32)],
    )(outT, av, bv, na, nb, diT)


def _tc_vhead(gsum, w1, b1, w2row, b2):
    def body(gs, w1_r, b1_r, w2_r, b2_r, vo):
        gv = gs[...] * (1.0 / N)
        t1 = jnp.maximum(_mm(gv, w1_r[...]) + b1_r[...], 0.0)
        t2 = jnp.sum(t1 * w2_r[...], axis=1, keepdims=True) + b2_r[...]
        vo[...] = jnp.tanh(t2)

    return pl.pallas_call(
        body,
        out_shape=jax.ShapeDtypeStruct((1, 1), _F32),
    )(gsum, w1, b1, w2row, b2)


def _tc_edge(gath, esrc, k, cvec, ab=None):
    ecols = esrc.shape[1]

    def make_body(with_norm):
        def body(*refs):
            if with_norm:
                g_r, e_r, k_r, c_r, a_r, b_r, co, so, qo = refs
                e = jnp.maximum(e_r[...] * a_r[...] + b_r[...], 0.0)
            else:
                g_r, e_r, k_r, c_r, co, so, qo = refs
                e = e_r[...]
            comb = g_r[...] + _mmd(e, k_r[...]) + c_r[...]
            co[...] = comb

            @pl.when(pl.program_id(0) == 0)
            def _init():
                so[...] = jnp.zeros_like(so)
                qo[...] = jnp.zeros_like(qo)

            so[...] += jnp.sum(comb, axis=0, keepdims=True)
            qo[...] += jnp.sum(comb * comb, axis=0, keepdims=True)
        return body

    in_specs = [
        pl.BlockSpec((EBLK, H), lambda i: (i, 0)),
        pl.BlockSpec((EBLK, ecols), lambda i: (i, 0)),
        pl.BlockSpec(k.shape, lambda i: (0, 0)),
        pl.BlockSpec((1, H), lambda i: (0, 0)),
    ]
    args = [gath, esrc, k, cvec]
    if ab is not None:
        in_specs += [pl.BlockSpec((1, H), lambda i: (0, 0))] * 2
        args += [ab[0], ab[1]]

    return pl.pallas_call(
        make_body(ab is not None),
        grid=(NEB,),
        in_specs=in_specs,
        out_specs=[pl.BlockSpec((EBLK, H), lambda i: (i, 0)),
                   pl.BlockSpec((1, H), lambda i: (0, 0)),
                   pl.BlockSpec((1, H), lambda i: (0, 0))],
        out_shape=[jax.ShapeDtypeStruct((E, H), _F32),
                   jax.ShapeDtypeStruct((1, H), _F32),
                   jax.ShapeDtypeStruct((1, H), _F32)],
    )(*args)


def _tc_head(comb, avec, bvec, w1, b1, w2row, b2):
    def body(c_r, a_r, b_r, w1_r, b1_r, w2_r, b2_r, so):
        e = jnp.maximum(c_r[...] * a_r[...] + b_r[...], 0.0)
        h = jnp.maximum(_mmd(e, w1_r[...]) + b1_r[...], 0.0)
        so[...] = (jnp.sum(h * w2_r[...], axis=1) + b2_r[0, 0]).reshape(1, 1, EBLK)

    return pl.pallas_call(
        body,
        grid=(NEB,),
        in_specs=[
            pl.BlockSpec((EBLK, H), lambda i: (i, 0)),
            pl.BlockSpec((1, H), lambda i: (0, 0)),
            pl.BlockSpec((1, H), lambda i: (0, 0)),
            pl.BlockSpec((H, 4 * H), lambda i: (0, 0)),
            pl.BlockSpec((1, 4 * H), lambda i: (0, 0)),
            pl.BlockSpec((1, 4 * H), lambda i: (0, 0)),
            pl.BlockSpec((1, 1), lambda i: (0, 0)),
        ],
        out_specs=pl.BlockSpec((1, 1, EBLK), lambda i: (i, 0, 0)),
        out_shape=jax.ShapeDtypeStruct((NEB, 1, EBLK), _F32),
    )(comb, avec, bvec, w1, b1, w2row, b2)


def _tc_softmax(scores2d):
    def body(s_r, p_r):
        s = s_r[...]
        m = jnp.max(s)
        ex = jnp.exp(s - m)
        p_r[...] = ex / jnp.sum(ex)

    return pl.pallas_call(
        body,
        out_shape=jax.ShapeDtypeStruct(scores2d.shape, _F32),
    )(scores2d)


def _fold(a, b):
    return jnp.dot(a, b, precision=_HI)


def kernel(edge_index, edge_attr, params):
    src = edge_index[0].astype(jnp.int32)
    dst = edge_index[1].astype(jnp.int32)
    p = params
    l0, l1 = p["layer0"], p["layer1"]

    cbwa0, cbwb0 = l0["cb_W"][:H], l0["cb_W"][H:]
    cbwa1, cbwb1 = l1["cb_W"][:H], l1["cb_W"][H:]
    na0 = _fold(l0["np_W"][:H], cbwa0)
    nb0 = _fold(l0["np_W"][H:], cbwa0)
    na1 = _fold(l1["np_W"][:H], cbwa1)
    nb1 = _fold(l1["np_W"][H:], cbwa1)
    k0 = _fold(l0["ep_W"], cbwb0)
    ek0 = _fold(p["ee_W"], k0)
    k1 = _fold(l1["ep_W"], cbwb1)
    c0 = (_fold(l0["np_b"], cbwa0) + _fold(p["ee_b"], k0)
          + _fold(l0["ep_b"], cbwb0) + l0["cb_b"]).reshape(1, H)
    c1 = (_fold(l1["np_b"], cbwa1) + _fold(l1["ep_b"], cbwb1)
          + l1["cb_b"]).reshape(1, H)
    x0row = (p["ne_W"][0] + p["ne_b"]).reshape(1, H)

    def _pad(t):
        return jnp.pad(t, ((0, 0), (0, TW - H)))

    def _bn_fold(s, q, cnt_n, gamma, betav):
        m = s / cnt_n
        var = jnp.maximum(q / cnt_n - m * m, 0.0)
        istd = lax.rsqrt(var + 1e-5)
        av = istd * gamma.reshape(1, H)
        return av, betav.reshape(1, H) - m * av

    cnt = _sc_count(dst)
    diT, y0T = _tc_prep(cnt, x0row, l0["gcn_W"])

    accT0 = _sc_colscat(y0T, src, dst)
    outT0, ns0, nq0 = _tc_node_a(accT0, y0T, diT, l0["gcn_b"].reshape(1, H))
    nav0, nbv0 = _bn_fold(ns0, nq0, N, l0["gcn_g"], l0["gcn_beta"])
    a0t, b0t, y1T, _ = _tc_node_b(outT0, nav0, nbv0, na0, nb0, diT,
                                  wn=l1["gcn_W"])

    accT1 = _sc_colscat(y1T, src, dst)
    gath0 = _sc_gather_pairs(_pad(a0t), _pad(b0t), src, dst)
    comb0, s0, q0 = _tc_edge(gath0, edge_attr, ek0, c0)

    outT1, ns1, nq1 = _tc_node_a(accT1, y1T, diT, l1["gcn_b"].reshape(1, H))
    nav1, nbv1 = _bn_fold(ns1, nq1, N, l1["gcn_g"], l1["gcn_beta"])
    a1t, b1t, g1 = _tc_node_b(outT1, nav1, nbv1, na1, nb1, diT)
    v = _tc_vhead(g1, p["vh_W1"], p["vh_b1"].reshape(1, H // 2),
                  p["vh_W2"].reshape(1, H // 2), p["vh_b2"].reshape(1, 1))

    av0, bv0 = _bn_fold(s0, q0, E, l0["eb_g"], l0["eb_beta"])
    gath1 = _sc_gather_pairs(_pad(a1t), _pad(b1t), src, dst)
    comb1, s1, q1 = _tc_edge(gath1, comb0, k1, c1, ab=(av0, bv0))

    av1, bv1 = _bn_fold(s1, q1, E, l1["eb_g"], l1["eb_beta"])
    scores = _tc_head(comb1, av1, bv1, p["ph_W1"], p["ph_b1"].reshape(1, 4 * H),
                      p["ph_W2"].reshape(1, 4 * H), p["ph_b2"].reshape(1, 1))
    policy = _tc_softmax(scores.reshape(2500, 128)).reshape(E)
    return policy, v.reshape(1)
